# baseline (speedup 1.0000x reference)
import functools

import jax
import jax.numpy as jnp
from jax import lax
from jax.experimental import pallas as pl
from jax.experimental.pallas import tpu as pltpu
from jax.experimental.pallas import tpu_sc as plsc

BATCH = 16384
HIST = 50
EMBED_DIM = 32

_G = 128
_NBLK = HIST * (BATCH // _G)
_NW = 32
_BPW = _NBLK // _NW
_TPH = BATCH // _G
_HROWS = 3


def _make_gather():
    mesh = plsc.VectorSubcoreMesh(core_axis_name="c", subcore_axis_name="s")

    @functools.partial(
        pl.kernel,
        mesh=mesh,
        out_type=jax.ShapeDtypeStruct((HIST * 4 * _TPH, 8 * _G), jnp.float32),
        scratch_types=[
            pltpu.VMEM((_HROWS, BATCH), jnp.int32),
            pltpu.VMEM((_G, EMBED_DIM), jnp.float32),
            pltpu.VMEM((_G, EMBED_DIM), jnp.float32),
            pltpu.VMEM((EMBED_DIM * _G,), jnp.float32),
            pltpu.VMEM((EMBED_DIM * _G,), jnp.float32),
            pltpu.SemaphoreType.DMA,
            pltpu.SemaphoreType.DMA,
            pltpu.SemaphoreType.DMA,
            pltpu.SemaphoreType.DMA,
        ],
        compiler_params=pltpu.CompilerParams(use_tc_tiling_on_sc=False,
                                             needs_layout_passes=False,
                                             disable_bounds_checks=True),
    )
    def gather_kernel(idx_hbm, table_hbm, out_hbm, idx_v,
                      rows0, rows1, t0, t1, gs0, gs1, os0, os1):
        rows = (rows0, rows1)
        tile = (t0, t1)
        gsem = (gs0, gs1)
        osem = (os0, os1)
        wid = lax.axis_index("s") * 2 + lax.axis_index("c")
        blk_base = wid * _BPW
        h0 = jnp.minimum(blk_base // _TPH, HIST - _HROWS)
        pltpu.sync_copy(idx_hbm.at[pl.ds(h0, _HROWS)], idx_v)

        lane = lax.iota(jnp.int32, 16)
        dv128 = [(lane + h * 16) * _G for h in range(2)]

        def fire_gather(i, rb):
            j = blk_base + i
            r = j // _TPH - h0
            off = (j % _TPH) * _G
            pltpu.async_copy(table_hbm.at[idx_v.at[r, pl.ds(off, _G)]],
                             rows[rb], gsem[rb])

        def drain_gather(rb):
            pltpu.make_async_copy(table_hbm.at[pl.ds(0, _G)],
                                  rows[rb], gsem[rb]).wait()

        def transpose(rb, tb):
            for b1 in range(_G):
                for h in range(2):
                    v = rows[rb][b1, pl.ds(h * 16, 16)]
                    plsc.store_scatter(tile[tb], [dv128[h] + b1], v)

        def fire_out(i, tb):
            j = blk_base + i
            row0 = (j // _TPH) * 512 + (j % _TPH)
            for dt in range(4):
                pltpu.async_copy(tile[tb].at[pl.ds(dt * 8 * _G, 8 * _G)],
                                 out_hbm.at[row0 + dt * _TPH], osem[tb])

        def wait_out(tb):
            for dt in range(4):
                pltpu.make_async_copy(tile[tb].at[pl.ds(dt * 8 * _G, 8 * _G)],
                                      out_hbm.at[0], osem[tb]).wait()

        fire_gather(0, 0)

        def body(i0):
            for b in range(2):
                i = i0 + b

                @pl.when(i + 1 < _BPW)
                def _():
                    fire_gather(i + 1, 1 - b)

                drain_gather(b)

                @pl.when(i >= 2)
                def _():
                    wait_out(b)

                transpose(b, b)
                fire_out(i, b)

        pl.loop(0, _BPW, step=2)(body)
        wait_out(0)
        wait_out(1)

    return gather_kernel


_gather = _make_gather()


def kernel(action, embedding):
    out5 = _gather(action.T, embedding)
    y = out5.reshape(HIST, 4, _TPH, 8, _G).transpose(2, 4, 0, 1, 3)
    return y.reshape(BATCH, HIST, EMBED_DIM)


# speedup vs baseline: 1.1685x; 1.1685x over previous
import functools

import jax
import jax.numpy as jnp
from jax import lax
from jax.experimental import pallas as pl
from jax.experimental.pallas import tpu as pltpu
from jax.experimental.pallas import tpu_sc as plsc

BATCH = 16384
HIST = 50
EMBED_DIM = 32

_G = 128
_NBLK = HIST * (BATCH // _G)
_NW = 32
_BPW = _NBLK // _NW
_TPH = BATCH // _G
_HROWS = 3


def _make_gather():
    mesh = plsc.VectorSubcoreMesh(core_axis_name="c", subcore_axis_name="s")

    @functools.partial(
        pl.kernel,
        mesh=mesh,
        out_type=jax.ShapeDtypeStruct((HIST * 4 * _TPH, 8 * _G), jnp.float32),
        scratch_types=[
            pltpu.VMEM((_HROWS, BATCH), jnp.int32),
            pltpu.VMEM((_G, EMBED_DIM), jnp.float32),
            pltpu.VMEM((_G, EMBED_DIM), jnp.float32),
            pltpu.VMEM((EMBED_DIM * _G,), jnp.float32),
            pltpu.VMEM((EMBED_DIM * _G,), jnp.float32),
            pltpu.SemaphoreType.DMA,
            pltpu.SemaphoreType.DMA,
            pltpu.SemaphoreType.DMA,
            pltpu.SemaphoreType.DMA,
        ],
        compiler_params=pltpu.CompilerParams(use_tc_tiling_on_sc=False,
                                             needs_layout_passes=False,
                                             disable_bounds_checks=True),
    )
    def gather_kernel(idx_hbm, table_hbm, out_hbm, idx_v,
                      rows0, rows1, t0, t1, gs0, gs1, os0, os1):
        rows = (rows0, rows1)
        tile = (t0, t1)
        gsem = (gs0, gs1)
        osem = (os0, os1)
        wid = lax.axis_index("s") * 2 + lax.axis_index("c")
        blk_base = wid * _BPW
        h0 = jnp.minimum(blk_base // _TPH, HIST - _HROWS)
        pltpu.sync_copy(idx_hbm.at[pl.ds(h0, _HROWS)], idx_v)

        lane = lax.iota(jnp.int32, 16)
        dv128 = [(lane + h * 16) * _G for h in range(2)]

        def fire_gather(i, rb):
            j = blk_base + i
            r = j // _TPH - h0
            off = (j % _TPH) * _G
            pltpu.async_copy(table_hbm.at[idx_v.at[r, pl.ds(off, _G)]],
                             rows[rb], gsem[rb])

        def drain_gather(rb):
            pltpu.make_async_copy(table_hbm.at[pl.ds(0, _G)],
                                  rows[rb], gsem[rb]).wait()

        def transpose(rb, tb):
            @plsc.parallel_loop(0, _G, step=1, unroll=8)
            def _(b1):
                for h in range(2):
                    v = rows[rb][b1, pl.ds(h * 16, 16)]
                    plsc.store_scatter(tile[tb], [dv128[h] + b1], v)

        def fire_out(i, tb):
            j = blk_base + i
            row0 = (j // _TPH) * 512 + (j % _TPH)
            for dt in range(4):
                pltpu.async_copy(tile[tb].at[pl.ds(dt * 8 * _G, 8 * _G)],
                                 out_hbm.at[row0 + dt * _TPH], osem[tb])

        def wait_out(tb):
            for dt in range(4):
                pltpu.make_async_copy(tile[tb].at[pl.ds(dt * 8 * _G, 8 * _G)],
                                      out_hbm.at[0], osem[tb]).wait()

        fire_gather(0, 0)

        def body(i0):
            for b in range(2):
                i = i0 + b

                @pl.when(i + 1 < _BPW)
                def _():
                    fire_gather(i + 1, 1 - b)

                drain_gather(b)

                @pl.when(i >= 2)
                def _():
                    wait_out(b)

                transpose(b, b)
                fire_out(i, b)

        pl.loop(0, _BPW, step=2)(body)
        wait_out(0)
        wait_out(1)

    return gather_kernel


_gather = _make_gather()


def kernel(action, embedding):
    out5 = _gather(action.T, embedding)
    y = out5.reshape(HIST, 4, _TPH, 8, _G).transpose(2, 4, 0, 1, 3)
    return y.reshape(BATCH, HIST, EMBED_DIM)


# speedup vs baseline: 1.1695x; 1.0009x over previous
import functools

import jax
import jax.numpy as jnp
from jax import lax
from jax.experimental import pallas as pl
from jax.experimental.pallas import tpu as pltpu
from jax.experimental.pallas import tpu_sc as plsc

BATCH = 16384
HIST = 50
EMBED_DIM = 32

_G = 128
_NBLK = HIST * (BATCH // _G)
_NW = 32
_BPW = _NBLK // _NW
_TPH = BATCH // _G
_HROWS = 3


def _make_gather():
    mesh = plsc.VectorSubcoreMesh(core_axis_name="c", subcore_axis_name="s")

    @functools.partial(
        pl.kernel,
        mesh=mesh,
        out_type=jax.ShapeDtypeStruct((HIST * 4 * _TPH, 8 * _G), jnp.float32),
        scratch_types=[
            pltpu.VMEM((_HROWS, BATCH), jnp.int32),
            pltpu.VMEM((_G, EMBED_DIM), jnp.float32),
            pltpu.VMEM((_G, EMBED_DIM), jnp.float32),
            pltpu.VMEM((EMBED_DIM * _G,), jnp.float32),
            pltpu.VMEM((EMBED_DIM * _G,), jnp.float32),
            pltpu.SemaphoreType.DMA,
            pltpu.SemaphoreType.DMA,
            pltpu.SemaphoreType.DMA,
            pltpu.SemaphoreType.DMA,
        ],
        compiler_params=pltpu.CompilerParams(use_tc_tiling_on_sc=False,
                                             needs_layout_passes=False,
                                             disable_bounds_checks=True),
    )
    def gather_kernel(idx_hbm, table_hbm, out_hbm, idx_v,
                      rows0, rows1, t0, t1, gs0, gs1, os0, os1):
        rows = (rows0, rows1)
        tile = (t0, t1)
        gsem = (gs0, gs1)
        osem = (os0, os1)
        wid = lax.axis_index("s") * 2 + lax.axis_index("c")
        blk_base = wid * _BPW
        h0 = jnp.minimum(blk_base // _TPH, HIST - _HROWS)
        pltpu.sync_copy(idx_hbm.at[pl.ds(h0, _HROWS)], idx_v)

        lane = lax.iota(jnp.int32, 16)
        dv128 = [(lane + h * 16) * _G for h in range(2)]

        def fire_gather(i, rb):
            j = blk_base + i
            r = j // _TPH - h0
            off = (j % _TPH) * _G
            pltpu.async_copy(table_hbm.at[idx_v.at[r, pl.ds(off, _G)]],
                             rows[rb], gsem[rb])

        def drain_gather(rb):
            pltpu.make_async_copy(table_hbm.at[pl.ds(0, _G)],
                                  rows[rb], gsem[rb]).wait()

        def transpose(rb, tb):
            @plsc.parallel_loop(0, _G, step=1, unroll=16)
            def _(b1):
                for h in range(2):
                    v = rows[rb][b1, pl.ds(h * 16, 16)]
                    plsc.store_scatter(tile[tb], [dv128[h] + b1], v)

        def fire_out(i, tb):
            j = blk_base + i
            row0 = (j // _TPH) * 512 + (j % _TPH)
            for dt in range(4):
                pltpu.async_copy(tile[tb].at[pl.ds(dt * 8 * _G, 8 * _G)],
                                 out_hbm.at[row0 + dt * _TPH], osem[tb])

        def wait_out(tb):
            for dt in range(4):
                pltpu.make_async_copy(tile[tb].at[pl.ds(dt * 8 * _G, 8 * _G)],
                                      out_hbm.at[0], osem[tb]).wait()

        fire_gather(0, 0)

        def body(i0):
            for b in range(2):
                i = i0 + b

                @pl.when(i + 1 < _BPW)
                def _():
                    fire_gather(i + 1, 1 - b)

                drain_gather(b)

                @pl.when(i >= 2)
                def _():
                    wait_out(b)

                transpose(b, b)
                fire_out(i, b)

        pl.loop(0, _BPW, step=2)(body)
        wait_out(0)
        wait_out(1)

    return gather_kernel


_gather = _make_gather()


def kernel(action, embedding):
    out5 = _gather(action.T, embedding)
    y = out5.reshape(HIST, 4, _TPH, 8, _G).transpose(2, 4, 0, 1, 3)
    return y.reshape(BATCH, HIST, EMBED_DIM)
